# Initial kernel scaffold; baseline (speedup 1.0000x reference)
import jax
import jax.numpy as jnp
from jax.experimental import pallas as pl


def kernel(X, edge_idx, edge_w, We, be, ge, bbe, Wn, bn, gn, bbn):
    raise NotImplementedError("write your pallas kernel here")


# speedup vs baseline: 19.4585x; 19.4585x over previous
import functools

import jax
import jax.numpy as jnp
from jax.experimental import pallas as pl
from jax.experimental.pallas import tpu as pltpu

E = 64


def _gelu_exact(x):
    return 0.5 * x * (1.0 + jax.lax.erf(x * 0.7071067811865476))


def _layer_norm(x, g, b, eps=1e-5):
    mu = jnp.mean(x, axis=-1, keepdims=True)
    var = jnp.mean((x - mu) ** 2, axis=-1, keepdims=True)
    return (x - mu) * jax.lax.rsqrt(var + eps) * g + b


def _dense_s(idx, w, nblk):
    k_dim = idx.shape[-1]
    iota = jax.lax.broadcasted_iota(jnp.int32, (nblk, E), 1)
    s = jnp.zeros((nblk, E), jnp.float32)
    for k in range(k_dim):
        s = s + jnp.where(idx[:, k : k + 1] == iota, w[:, k : k + 1], 0.0)
    return s


def _he_kernel(idx_ref, w_ref, x_ref, he_ref, *, nblk):
    nb = pl.program_id(1)
    s = _dense_s(idx_ref[0], w_ref[0], nblk)
    acc = jax.lax.dot_general(
        s, x_ref[0], (((0,), (0,)), ((), ())), preferred_element_type=jnp.float32
    )

    @pl.when(nb == 0)
    def _():
        he_ref[0] = acc

    @pl.when(nb != 0)
    def _():
        he_ref[0] = he_ref[0] + acc


def _out_kernel(
    idx_ref, w_ref, x_ref, he_ref, we_ref, be_ref, ge_ref, bbe_ref,
    wn_ref, bn_ref, gn_ref, bbn_ref, o_ref, hep_ref, *, nblk,
):
    nb = pl.program_id(1)

    @pl.when(nb == 0)
    def _():
        h = (
            jnp.dot(he_ref[0], we_ref[...], preferred_element_type=jnp.float32)
            + be_ref[...]
        )
        hep_ref[...] = _layer_norm(_gelu_exact(h), ge_ref[...], bbe_ref[...])

    s = _dense_s(idx_ref[0], w_ref[0], nblk)
    y = jnp.dot(s, hep_ref[...], preferred_element_type=jnp.float32)
    z = jnp.dot(y, wn_ref[...], preferred_element_type=jnp.float32) + bn_ref[...]
    z = _layer_norm(_gelu_exact(z), gn_ref[...], bbn_ref[...])
    o_ref[0] = z + x_ref[0]


def kernel(X, edge_idx, edge_w, We, be, ge, bbe, Wn, bn, gn, bbn):
    B, N, D = X.shape
    K = edge_idx.shape[-1]
    nblk = min(512, N)
    n_blocks = N // nblk
    grid = (B, n_blocks)

    idx = edge_idx.astype(jnp.int32)
    w = edge_w.astype(jnp.float32)
    be2, ge2, bbe2 = be.reshape(1, D), ge.reshape(1, D), bbe.reshape(1, D)
    bn2, gn2, bbn2 = bn.reshape(1, D), gn.reshape(1, D), bbn.reshape(1, D)

    blk_idx = pl.BlockSpec((1, nblk, K), lambda b, i: (b, i, 0))
    blk_w = pl.BlockSpec((1, nblk, K), lambda b, i: (b, i, 0))
    blk_x = pl.BlockSpec((1, nblk, D), lambda b, i: (b, i, 0))
    blk_he = pl.BlockSpec((1, E, D), lambda b, i: (b, 0, 0))
    blk_dd = pl.BlockSpec((D, D), lambda b, i: (0, 0))
    blk_1d = pl.BlockSpec((1, D), lambda b, i: (0, 0))

    he = pl.pallas_call(
        functools.partial(_he_kernel, nblk=nblk),
        grid=grid,
        in_specs=[blk_idx, blk_w, blk_x],
        out_specs=blk_he,
        out_shape=jax.ShapeDtypeStruct((B, E, D), jnp.float32),
        compiler_params=pltpu.CompilerParams(
            dimension_semantics=("arbitrary", "arbitrary")
        ),
    )(idx, w, X)

    out = pl.pallas_call(
        functools.partial(_out_kernel, nblk=nblk),
        grid=grid,
        in_specs=[blk_idx, blk_w, blk_x, blk_he, blk_dd, blk_1d, blk_1d,
                  blk_1d, blk_dd, blk_1d, blk_1d, blk_1d],
        out_specs=blk_x,
        out_shape=jax.ShapeDtypeStruct((B, N, D), jnp.float32),
        scratch_shapes=[pltpu.VMEM((E, D), jnp.float32)],
        compiler_params=pltpu.CompilerParams(
            dimension_semantics=("arbitrary", "arbitrary")
        ),
    )(idx, w, X, he, We, be2, ge2, bbe2, Wn, bn2, gn2, bbn2)
    return out


# speedup vs baseline: 24.5777x; 1.2631x over previous
import functools

import jax
import jax.numpy as jnp
from jax.experimental import pallas as pl
from jax.experimental.pallas import tpu as pltpu

E = 64


def _gelu_exact(x):
    return 0.5 * x * (1.0 + jax.lax.erf(x * 0.7071067811865476))


def _layer_norm(x, g, b, eps=1e-5):
    mu = jnp.mean(x, axis=-1, keepdims=True)
    var = jnp.mean((x - mu) ** 2, axis=-1, keepdims=True)
    return (x - mu) * jax.lax.rsqrt(var + eps) * g + b


def _dense_s(idx, w, nblk):
    k_dim = idx.shape[-1]
    iota = jax.lax.broadcasted_iota(jnp.int32, (nblk, E), 1)
    s = jnp.zeros((nblk, E), jnp.float32)
    for k in range(k_dim):
        s = s + jnp.where(idx[:, k : k + 1] == iota, w[:, k : k + 1], 0.0)
    return s


def _he_kernel(
    idx_ref, w_ref, x_ref, we_ref, be_ref, ge_ref, bbe_ref, s_ref, he_ref,
    *, nblk, n_blocks,
):
    nb = pl.program_id(1)
    s = _dense_s(idx_ref[0], w_ref[0], nblk)
    s_ref[0] = s
    acc = jax.lax.dot_general(
        s, x_ref[0], (((0,), (0,)), ((), ())), preferred_element_type=jnp.float32
    )

    @pl.when(nb == 0)
    def _():
        he_ref[0] = acc

    @pl.when(nb != 0)
    def _():
        he_ref[0] = he_ref[0] + acc

    @pl.when(nb == n_blocks - 1)
    def _():
        h = (
            jnp.dot(he_ref[0], we_ref[...], preferred_element_type=jnp.float32)
            + be_ref[...]
        )
        he_ref[0] = _layer_norm(_gelu_exact(h), ge_ref[...], bbe_ref[...])


def _out_kernel(s_ref, x_ref, hep_ref, wn_ref, bn_ref, gn_ref, bbn_ref, o_ref):
    y = jnp.dot(s_ref[0], hep_ref[0], preferred_element_type=jnp.float32)
    z = jnp.dot(y, wn_ref[...], preferred_element_type=jnp.float32) + bn_ref[...]
    z = _layer_norm(_gelu_exact(z), gn_ref[...], bbn_ref[...])
    o_ref[0] = z + x_ref[0]


def kernel(X, edge_idx, edge_w, We, be, ge, bbe, Wn, bn, gn, bbn):
    B, N, D = X.shape
    K = edge_idx.shape[-1]
    nblk = min(512, N)
    n_blocks = N // nblk
    grid = (B, n_blocks)

    idx = edge_idx.astype(jnp.int32)
    w = edge_w.astype(jnp.float32)
    be2, ge2, bbe2 = be.reshape(1, D), ge.reshape(1, D), bbe.reshape(1, D)
    bn2, gn2, bbn2 = bn.reshape(1, D), gn.reshape(1, D), bbn.reshape(1, D)

    blk_idx = pl.BlockSpec((1, nblk, K), lambda b, i: (b, i, 0))
    blk_x = pl.BlockSpec((1, nblk, D), lambda b, i: (b, i, 0))
    blk_s = pl.BlockSpec((1, nblk, E), lambda b, i: (b, i, 0))
    blk_he = pl.BlockSpec((1, E, D), lambda b, i: (b, 0, 0))
    blk_dd = pl.BlockSpec((D, D), lambda b, i: (0, 0))
    blk_1d = pl.BlockSpec((1, D), lambda b, i: (0, 0))

    s_mat, hep = pl.pallas_call(
        functools.partial(_he_kernel, nblk=nblk, n_blocks=n_blocks),
        grid=grid,
        in_specs=[blk_idx, blk_idx, blk_x, blk_dd, blk_1d, blk_1d, blk_1d],
        out_specs=[blk_s, blk_he],
        out_shape=[
            jax.ShapeDtypeStruct((B, N, E), jnp.float32),
            jax.ShapeDtypeStruct((B, E, D), jnp.float32),
        ],
        compiler_params=pltpu.CompilerParams(
            dimension_semantics=("arbitrary", "arbitrary")
        ),
    )(idx, w, X, We, be2, ge2, bbe2)

    out = pl.pallas_call(
        _out_kernel,
        grid=grid,
        in_specs=[blk_s, blk_x, blk_he, blk_dd, blk_1d, blk_1d, blk_1d],
        out_specs=blk_x,
        out_shape=jax.ShapeDtypeStruct((B, N, D), jnp.float32),
        compiler_params=pltpu.CompilerParams(
            dimension_semantics=("arbitrary", "arbitrary")
        ),
    )(s_mat, X, hep, Wn, bn2, gn2, bbn2)
    return out


# speedup vs baseline: 26.6900x; 1.0859x over previous
import functools

import jax
import jax.numpy as jnp
from jax.experimental import pallas as pl
from jax.experimental.pallas import tpu as pltpu

E = 64


def _gelu_exact(x):
    return 0.5 * x * (1.0 + jax.lax.erf(x * 0.7071067811865476))


def _layer_norm(x, g, b, eps=1e-5):
    mu = jnp.mean(x, axis=-1, keepdims=True)
    var = jnp.mean((x - mu) ** 2, axis=-1, keepdims=True)
    return (x - mu) * jax.lax.rsqrt(var + eps) * g + b


def _dense_s(idx, w, nblk):
    k_dim = idx.shape[-1]
    iota = jax.lax.broadcasted_iota(jnp.int32, (nblk, E), 1)
    s = jnp.zeros((nblk, E), jnp.float32)
    for k in range(k_dim):
        s = s + jnp.where(idx[:, k : k + 1] == iota, w[:, k : k + 1], 0.0)
    return s


def _he_kernel(
    idx_ref, w_ref, x_ref, we_ref, be_ref, ge_ref, bbe_ref, s_ref, he_ref,
    *, nblk, n_blocks,
):
    nb = pl.program_id(1)
    s = _dense_s(idx_ref[0], w_ref[0], nblk)
    s_ref[0] = s
    acc = jax.lax.dot_general(
        s, x_ref[0], (((0,), (0,)), ((), ())), preferred_element_type=jnp.float32
    )

    @pl.when(nb == 0)
    def _():
        he_ref[0] = acc

    @pl.when(nb != 0)
    def _():
        he_ref[0] = he_ref[0] + acc

    @pl.when(nb == n_blocks - 1)
    def _():
        h = (
            jnp.dot(he_ref[0], we_ref[...], preferred_element_type=jnp.float32)
            + be_ref[...]
        )
        he_ref[0] = _layer_norm(_gelu_exact(h), ge_ref[...], bbe_ref[...])


def _out_kernel(s_ref, x_ref, hep_ref, wn_ref, bn_ref, gn_ref, bbn_ref, o_ref):
    y = jnp.dot(s_ref[0], hep_ref[0], preferred_element_type=jnp.float32)
    z = jnp.dot(y, wn_ref[...], preferred_element_type=jnp.float32) + bn_ref[...]
    z = _layer_norm(_gelu_exact(z), gn_ref[...], bbn_ref[...])
    o_ref[0] = z + x_ref[0]


def kernel(X, edge_idx, edge_w, We, be, ge, bbe, Wn, bn, gn, bbn):
    B, N, D = X.shape
    K = edge_idx.shape[-1]
    nblk = min(1024, N)
    n_blocks = N // nblk
    grid = (B, n_blocks)

    idx = edge_idx.astype(jnp.int32)
    w = edge_w.astype(jnp.float32)
    be2, ge2, bbe2 = be.reshape(1, D), ge.reshape(1, D), bbe.reshape(1, D)
    bn2, gn2, bbn2 = bn.reshape(1, D), gn.reshape(1, D), bbn.reshape(1, D)

    blk_idx = pl.BlockSpec((1, nblk, K), lambda b, i: (b, i, 0))
    blk_x = pl.BlockSpec((1, nblk, D), lambda b, i: (b, i, 0))
    blk_s = pl.BlockSpec((1, nblk, E), lambda b, i: (b, i, 0))
    blk_he = pl.BlockSpec((1, E, D), lambda b, i: (b, 0, 0))
    blk_dd = pl.BlockSpec((D, D), lambda b, i: (0, 0))
    blk_1d = pl.BlockSpec((1, D), lambda b, i: (0, 0))

    s_mat, hep = pl.pallas_call(
        functools.partial(_he_kernel, nblk=nblk, n_blocks=n_blocks),
        grid=grid,
        in_specs=[blk_idx, blk_idx, blk_x, blk_dd, blk_1d, blk_1d, blk_1d],
        out_specs=[blk_s, blk_he],
        out_shape=[
            jax.ShapeDtypeStruct((B, N, E), jnp.float32),
            jax.ShapeDtypeStruct((B, E, D), jnp.float32),
        ],
        compiler_params=pltpu.CompilerParams(
            dimension_semantics=("arbitrary", "arbitrary")
        ),
    )(idx, w, X, We, be2, ge2, bbe2)

    out = pl.pallas_call(
        _out_kernel,
        grid=grid,
        in_specs=[blk_s, blk_x, blk_he, blk_dd, blk_1d, blk_1d, blk_1d],
        out_specs=blk_x,
        out_shape=jax.ShapeDtypeStruct((B, N, D), jnp.float32),
        compiler_params=pltpu.CompilerParams(
            dimension_semantics=("arbitrary", "arbitrary")
        ),
    )(s_mat, X, hep, Wn, bn2, gn2, bbn2)
    return out
